# Initial kernel scaffold; baseline (speedup 1.0000x reference)
import jax
import jax.numpy as jnp
from jax.experimental import pallas as pl


def kernel(x, edge_index, W1, b1, W2, b2, sigma1, sigma2):
    raise NotImplementedError("write your pallas kernel here")


# speedup vs baseline: 30.3446x; 30.3446x over previous
import functools

import jax
import jax.numpy as jnp
from jax import lax
from jax.experimental import pallas as pl
from jax.experimental.pallas import tpu as pltpu
from jax.experimental.pallas import tpu_sc as plsc

N = 10000
F_IN = 128
NHID = 64
NCLS = 40
NCLSP = 48
E = 320000

NC = 2
NS = 16
NW = NC * NS
CHUNK = 128
CPT = -(-E // (NW * CHUNK))
E_PAD = CPT * NW * CHUNK
N_PAD = 10240
ROWS_PER_TILE = N_PAD // NS
R_TC = 1024

_MESH = dict(core_axis_name="c", subcore_axis_name="s", num_cores=NC,
             num_subcores=NS)


@functools.cache
def _make_edge_agg(F):
    mesh = plsc.VectorSubcoreMesh(**_MESH)

    @functools.partial(
        pl.kernel,
        out_type=jax.ShapeDtypeStruct((NC, N_PAD, F), jnp.float32),
        mesh=mesh,
        compiler_params=pltpu.CompilerParams(use_tc_tiling_on_sc=False),
        scratch_types=[
            pltpu.VMEM((CPT, CHUNK), jnp.int32),
            pltpu.VMEM((CPT, CHUNK), jnp.int32),
            pltpu.VMEM((CHUNK, F), jnp.float32),
            pltpu.VMEM_SHARED((N_PAD, F), jnp.float32),
            pltpu.VMEM_SHARED((N_PAD, F), jnp.float32),
            pltpu.SemaphoreType.DMA,
        ],
    )
    def edge_agg(src_hbm, dst_hbm, table_hbm, zeros_hbm, out_hbm,
                 srcv, dstv, buf, acc, tbl, sem):
        cid = lax.axis_index("c")
        sid = lax.axis_index("s")
        w = cid * NS + sid
        pltpu.sync_copy(src_hbm.at[w], srcv)
        pltpu.sync_copy(dst_hbm.at[w], dstv)
        sl = pl.ds(sid * ROWS_PER_TILE, ROWS_PER_TILE)
        pltpu.sync_copy(table_hbm.at[sl], tbl.at[sl])
        pltpu.sync_copy(zeros_hbm, acc.at[sl])
        plsc.subcore_barrier()

        @pl.loop(0, CPT)
        def _(j):
            pltpu.async_copy(tbl.at[srcv.at[j]], buf, sem).wait()
            pltpu.sync_copy(buf, acc.at[dstv.at[j]], add=True)

        plsc.subcore_barrier()
        pltpu.sync_copy(acc.at[sl], out_hbm.at[cid, sl])

    return edge_agg


@functools.cache
def _make_deg():
    mesh = plsc.VectorSubcoreMesh(**_MESH)

    @functools.partial(
        pl.kernel,
        out_type=jax.ShapeDtypeStruct((NC, N_PAD, 16), jnp.float32),
        mesh=mesh,
        compiler_params=pltpu.CompilerParams(use_tc_tiling_on_sc=False),
        scratch_types=[
            pltpu.VMEM((CPT, CHUNK), jnp.int32),
            pltpu.VMEM((CHUNK, 16), jnp.float32),
            pltpu.VMEM_SHARED((N_PAD, 16), jnp.float32),
        ],
    )
    def deg(dst_hbm, zeros_hbm, out_hbm, dstv, buf, acc):
        cid = lax.axis_index("c")
        sid = lax.axis_index("s")
        w = cid * NS + sid
        pltpu.sync_copy(dst_hbm.at[w], dstv)
        one = jnp.ones((16,), jnp.float32)

        @pl.loop(0, CHUNK)
        def _(r):
            buf[r] = one

        pltpu.sync_copy(zeros_hbm, acc.at[pl.ds(sid * ROWS_PER_TILE,
                                                ROWS_PER_TILE)])
        plsc.subcore_barrier()

        @pl.loop(0, CPT)
        def _(j):
            pltpu.sync_copy(buf, acc.at[dstv.at[j]], add=True)

        plsc.subcore_barrier()
        sl = pl.ds(sid * ROWS_PER_TILE, ROWS_PER_TILE)
        pltpu.sync_copy(acc.at[sl], out_hbm.at[cid, sl])

    return deg


def _t1_body(x_ref, w_ref, degp_ref, hs_ref, dinv_ref):
    degsum = degp_ref[0] + degp_ref[1]
    dinv = lax.rsqrt(degsum + 1.0)
    dinv_ref[...] = dinv
    h = jnp.dot(x_ref[...], w_ref[...], preferred_element_type=jnp.float32)
    hs_ref[...] = h * dinv[:, :1]


def _t1(xp, W1, degp):
    grid = (N_PAD // R_TC,)
    return pl.pallas_call(
        _t1_body,
        grid=grid,
        in_specs=[
            pl.BlockSpec((R_TC, F_IN), lambda i: (i, 0)),
            pl.BlockSpec((F_IN, NHID), lambda i: (0, 0)),
            pl.BlockSpec((NC, R_TC, 16), lambda i: (0, i, 0)),
        ],
        out_specs=[
            pl.BlockSpec((R_TC, NHID), lambda i: (i, 0)),
            pl.BlockSpec((R_TC, 16), lambda i: (i, 0)),
        ],
        out_shape=[
            jax.ShapeDtypeStruct((N_PAD, NHID), jnp.float32),
            jax.ShapeDtypeStruct((N_PAD, 16), jnp.float32),
        ],
    )(xp, W1, degp)


def _t2_body(agg_ref, hs_ref, dinv_ref, w_ref, b_ref, out_ref):
    a = agg_ref[0] + agg_ref[1] + hs_ref[...]
    dinv = dinv_ref[:, :1]
    out1 = jnp.maximum(dinv * a + b_ref[...], 0.0)
    h2 = jnp.dot(out1, w_ref[...], preferred_element_type=jnp.float32)
    out_ref[...] = h2 * dinv


def _t2(agg1, hs1, dinv, W2p, b1r):
    grid = (N_PAD // R_TC,)
    return pl.pallas_call(
        _t2_body,
        grid=grid,
        in_specs=[
            pl.BlockSpec((NC, R_TC, NHID), lambda i: (0, i, 0)),
            pl.BlockSpec((R_TC, NHID), lambda i: (i, 0)),
            pl.BlockSpec((R_TC, 16), lambda i: (i, 0)),
            pl.BlockSpec((NHID, NCLSP), lambda i: (0, 0)),
            pl.BlockSpec((1, NHID), lambda i: (0, 0)),
        ],
        out_specs=pl.BlockSpec((R_TC, NCLSP), lambda i: (i, 0)),
        out_shape=jax.ShapeDtypeStruct((N_PAD, NCLSP), jnp.float32),
    )(agg1, hs1, dinv, W2p, b1r)


def _t3_body(agg_ref, hs_ref, dinv_ref, b_ref, out_ref):
    a = agg_ref[0] + agg_ref[1] + hs_ref[...]
    logits = dinv_ref[:, :1] * a + b_ref[...]
    col = lax.broadcasted_iota(jnp.int32, (R_TC, NCLSP), 1)
    logits = jnp.where(col < NCLS, logits, -jnp.inf)
    m = jnp.max(logits, axis=1, keepdims=True)
    ex = jnp.exp(logits - m)
    s = jnp.sum(ex, axis=1, keepdims=True)
    res = (logits - m) - jnp.log(s)
    out_ref[...] = res[:, :NCLS]


def _t3(agg2, hs2, dinv, b2r):
    grid = (N_PAD // R_TC,)
    return pl.pallas_call(
        _t3_body,
        grid=grid,
        in_specs=[
            pl.BlockSpec((NC, R_TC, NCLSP), lambda i: (0, i, 0)),
            pl.BlockSpec((R_TC, NCLSP), lambda i: (i, 0)),
            pl.BlockSpec((R_TC, 16), lambda i: (i, 0)),
            pl.BlockSpec((1, NCLSP), lambda i: (0, 0)),
        ],
        out_specs=pl.BlockSpec((R_TC, NCLS), lambda i: (i, 0)),
        out_shape=jax.ShapeDtypeStruct((N_PAD, NCLS), jnp.float32),
    )(agg2, hs2, dinv, b2r)


def kernel(x, edge_index, W1, b1, W2, b2, sigma1, sigma2):
    del sigma1, sigma2
    src = edge_index[0].astype(jnp.int32)
    dst = edge_index[1].astype(jnp.int32)
    npad_e = E_PAD - E
    pad_idx = N + (jnp.arange(npad_e, dtype=jnp.int32) % (N_PAD - N))
    srcp = jnp.concatenate([src, pad_idx]).reshape(NW, CPT, CHUNK)
    dstp = jnp.concatenate([dst, pad_idx]).reshape(NW, CPT, CHUNK)
    xp = jnp.pad(x, ((0, N_PAD - N), (0, 0)))
    W2p = jnp.pad(W2, ((0, 0), (0, NCLSP - NCLS)))
    b1r = b1.reshape(1, NHID)
    b2r = jnp.pad(b2, (0, NCLSP - NCLS)).reshape(1, NCLSP)
    zeros16 = jnp.zeros((ROWS_PER_TILE, 16), jnp.float32)
    zeros64 = jnp.zeros((ROWS_PER_TILE, NHID), jnp.float32)
    zeros48 = jnp.zeros((ROWS_PER_TILE, NCLSP), jnp.float32)
    degp = _make_deg()(dstp, zeros16)
    hs1, dinv = _t1(xp, W1, degp)
    agg1 = _make_edge_agg(NHID)(srcp, dstp, hs1, zeros64)
    hs2 = _t2(agg1, hs1, dinv, W2p, b1r)
    agg2 = _make_edge_agg(NCLSP)(srcp, dstp, hs2, zeros48)
    out = _t3(agg2, hs2, dinv, b2r)
    return out[:N]


# speedup vs baseline: 35.4985x; 1.1698x over previous
import functools

import jax
import jax.numpy as jnp
from jax import lax
from jax.experimental import pallas as pl
from jax.experimental.pallas import tpu as pltpu
from jax.experimental.pallas import tpu_sc as plsc

N = 10000
F_IN = 128
NHID = 64
NCLS = 40
NCLSP = 48
E = 320000

NC = 2
NS = 16
NW = NC * NS
CHUNK = 128
CPT = 2 * (-(-E // (2 * NW * CHUNK)))
E_PAD = CPT * NW * CHUNK
N_PAD = 10240
ROWS_PER_TILE = N_PAD // NS
R_TC = 1024

_MESH = dict(core_axis_name="c", subcore_axis_name="s", num_cores=NC,
             num_subcores=NS)


@functools.cache
def _make_edge_agg(F):
    mesh = plsc.VectorSubcoreMesh(**_MESH)

    @functools.partial(
        pl.kernel,
        out_type=jax.ShapeDtypeStruct((NC, N_PAD, F), jnp.float32),
        mesh=mesh,
        compiler_params=pltpu.CompilerParams(use_tc_tiling_on_sc=False),
        scratch_types=[
            pltpu.VMEM((CPT, CHUNK), jnp.int32),
            pltpu.VMEM((CPT, CHUNK), jnp.int32),
            pltpu.VMEM((CHUNK, F), jnp.float32),
            pltpu.VMEM((CHUNK, F), jnp.float32),
            pltpu.VMEM_SHARED((N_PAD, F), jnp.float32),
            pltpu.VMEM_SHARED((N_PAD, F), jnp.float32),
            pltpu.SemaphoreType.DMA,
            pltpu.SemaphoreType.DMA,
        ],
    )
    def edge_agg(src_hbm, dst_hbm, table_hbm, zeros_hbm, out_hbm,
                 srcv, dstv, buf0, buf1, acc, tbl, gsem0, gsem1):
        cid = lax.axis_index("c")
        sid = lax.axis_index("s")
        w = cid * NS + sid
        pltpu.sync_copy(src_hbm.at[w], srcv)
        pltpu.sync_copy(dst_hbm.at[w], dstv)
        sl = pl.ds(sid * ROWS_PER_TILE, ROWS_PER_TILE)
        pltpu.sync_copy(table_hbm.at[sl], tbl.at[sl])
        pltpu.sync_copy(zeros_hbm, acc.at[sl])
        plsc.subcore_barrier()
        pltpu.async_copy(tbl.at[srcv.at[0]], buf0, gsem0)

        @pl.loop(0, CPT // 2)
        def _(g):
            j0 = g * 2
            pltpu.make_async_copy(tbl.at[srcv.at[j0]], buf0, gsem0).wait()
            pltpu.async_copy(tbl.at[srcv.at[j0 + 1]], buf1, gsem1)
            pltpu.sync_copy(buf0, acc.at[dstv.at[j0]], add=True)
            pltpu.make_async_copy(tbl.at[srcv.at[j0 + 1]], buf1, gsem1).wait()

            @pl.when(j0 + 2 < CPT)
            def _():
                pltpu.async_copy(tbl.at[srcv.at[j0 + 2]], buf0, gsem0)

            pltpu.sync_copy(buf1, acc.at[dstv.at[j0 + 1]], add=True)

        plsc.subcore_barrier()
        pltpu.sync_copy(acc.at[sl], out_hbm.at[cid, sl])

    return edge_agg


@functools.cache
def _make_deg():
    mesh = plsc.VectorSubcoreMesh(**_MESH)

    @functools.partial(
        pl.kernel,
        out_type=jax.ShapeDtypeStruct((NC, N_PAD, 16), jnp.float32),
        mesh=mesh,
        compiler_params=pltpu.CompilerParams(use_tc_tiling_on_sc=False),
        scratch_types=[
            pltpu.VMEM((CPT, CHUNK), jnp.int32),
            pltpu.VMEM((CHUNK, 16), jnp.float32),
            pltpu.VMEM_SHARED((N_PAD, 16), jnp.float32),
        ],
    )
    def deg(dst_hbm, zeros_hbm, out_hbm, dstv, buf, acc):
        cid = lax.axis_index("c")
        sid = lax.axis_index("s")
        w = cid * NS + sid
        pltpu.sync_copy(dst_hbm.at[w], dstv)
        one = jnp.ones((16,), jnp.float32)

        @pl.loop(0, CHUNK)
        def _(r):
            buf[r] = one

        pltpu.sync_copy(zeros_hbm, acc.at[pl.ds(sid * ROWS_PER_TILE,
                                                ROWS_PER_TILE)])
        plsc.subcore_barrier()

        @pl.loop(0, CPT)
        def _(j):
            pltpu.sync_copy(buf, acc.at[dstv.at[j]], add=True)

        plsc.subcore_barrier()
        sl = pl.ds(sid * ROWS_PER_TILE, ROWS_PER_TILE)
        pltpu.sync_copy(acc.at[sl], out_hbm.at[cid, sl])

    return deg


def _t1_body(x_ref, w_ref, degp_ref, hs_ref, dinv_ref):
    degsum = degp_ref[0] + degp_ref[1]
    dinv = lax.rsqrt(degsum + 1.0)
    dinv_ref[...] = dinv
    h = jnp.dot(x_ref[...], w_ref[...], preferred_element_type=jnp.float32)
    hs_ref[...] = h * dinv[:, :1]


def _t1(xp, W1, degp):
    grid = (N_PAD // R_TC,)
    return pl.pallas_call(
        _t1_body,
        grid=grid,
        in_specs=[
            pl.BlockSpec((R_TC, F_IN), lambda i: (i, 0)),
            pl.BlockSpec((F_IN, NHID), lambda i: (0, 0)),
            pl.BlockSpec((NC, R_TC, 16), lambda i: (0, i, 0)),
        ],
        out_specs=[
            pl.BlockSpec((R_TC, NHID), lambda i: (i, 0)),
            pl.BlockSpec((R_TC, 16), lambda i: (i, 0)),
        ],
        out_shape=[
            jax.ShapeDtypeStruct((N_PAD, NHID), jnp.float32),
            jax.ShapeDtypeStruct((N_PAD, 16), jnp.float32),
        ],
    )(xp, W1, degp)


def _t2_body(agg_ref, hs_ref, dinv_ref, w_ref, b_ref, out_ref):
    a = agg_ref[0] + agg_ref[1] + hs_ref[...]
    dinv = dinv_ref[:, :1]
    out1 = jnp.maximum(dinv * a + b_ref[...], 0.0)
    h2 = jnp.dot(out1, w_ref[...], preferred_element_type=jnp.float32)
    out_ref[...] = h2 * dinv


def _t2(agg1, hs1, dinv, W2p, b1r):
    grid = (N_PAD // R_TC,)
    return pl.pallas_call(
        _t2_body,
        grid=grid,
        in_specs=[
            pl.BlockSpec((NC, R_TC, NHID), lambda i: (0, i, 0)),
            pl.BlockSpec((R_TC, NHID), lambda i: (i, 0)),
            pl.BlockSpec((R_TC, 16), lambda i: (i, 0)),
            pl.BlockSpec((NHID, NCLSP), lambda i: (0, 0)),
            pl.BlockSpec((1, NHID), lambda i: (0, 0)),
        ],
        out_specs=pl.BlockSpec((R_TC, NCLSP), lambda i: (i, 0)),
        out_shape=jax.ShapeDtypeStruct((N_PAD, NCLSP), jnp.float32),
    )(agg1, hs1, dinv, W2p, b1r)


def _t3_body(agg_ref, hs_ref, dinv_ref, b_ref, out_ref):
    a = agg_ref[0] + agg_ref[1] + hs_ref[...]
    logits = dinv_ref[:, :1] * a + b_ref[...]
    col = lax.broadcasted_iota(jnp.int32, (R_TC, NCLSP), 1)
    logits = jnp.where(col < NCLS, logits, -jnp.inf)
    m = jnp.max(logits, axis=1, keepdims=True)
    ex = jnp.exp(logits - m)
    s = jnp.sum(ex, axis=1, keepdims=True)
    res = (logits - m) - jnp.log(s)
    out_ref[...] = res[:, :NCLS]


def _t3(agg2, hs2, dinv, b2r):
    grid = (N_PAD // R_TC,)
    return pl.pallas_call(
        _t3_body,
        grid=grid,
        in_specs=[
            pl.BlockSpec((NC, R_TC, NCLSP), lambda i: (0, i, 0)),
            pl.BlockSpec((R_TC, NCLSP), lambda i: (i, 0)),
            pl.BlockSpec((R_TC, 16), lambda i: (i, 0)),
            pl.BlockSpec((1, NCLSP), lambda i: (0, 0)),
        ],
        out_specs=pl.BlockSpec((R_TC, NCLS), lambda i: (i, 0)),
        out_shape=jax.ShapeDtypeStruct((N_PAD, NCLS), jnp.float32),
    )(agg2, hs2, dinv, b2r)


def kernel(x, edge_index, W1, b1, W2, b2, sigma1, sigma2):
    del sigma1, sigma2
    src = edge_index[0].astype(jnp.int32)
    dst = edge_index[1].astype(jnp.int32)
    npad_e = E_PAD - E
    pad_idx = N + (jnp.arange(npad_e, dtype=jnp.int32) % (N_PAD - N))
    srcp = jnp.concatenate([src, pad_idx]).reshape(NW, CPT, CHUNK)
    dstp = jnp.concatenate([dst, pad_idx]).reshape(NW, CPT, CHUNK)
    xp = jnp.pad(x, ((0, N_PAD - N), (0, 0)))
    W2p = jnp.pad(W2, ((0, 0), (0, NCLSP - NCLS)))
    b1r = b1.reshape(1, NHID)
    b2r = jnp.pad(b2, (0, NCLSP - NCLS)).reshape(1, NCLSP)
    zeros16 = jnp.zeros((ROWS_PER_TILE, 16), jnp.float32)
    zeros64 = jnp.zeros((ROWS_PER_TILE, NHID), jnp.float32)
    zeros48 = jnp.zeros((ROWS_PER_TILE, NCLSP), jnp.float32)
    degp = _make_deg()(dstp, zeros16)
    hs1, dinv = _t1(xp, W1, degp)
    agg1 = _make_edge_agg(NHID)(srcp, dstp, hs1, zeros64)
    hs2 = _t2(agg1, hs1, dinv, W2p, b1r)
    agg2 = _make_edge_agg(NCLSP)(srcp, dstp, hs2, zeros48)
    out = _t3(agg2, hs2, dinv, b2r)
    return out[:N]
